# baseline (speedup 1.0000x reference)
import functools

import jax
import jax.numpy as jnp
from jax import lax
from jax.experimental import pallas as pl
from jax.experimental.pallas import tpu as pltpu
from jax.experimental.pallas import tpu_sc as plsc

NUM_CONTEXTS = 1000000
EMB_DIM = 32
BATCH = 16384

_info = plsc.get_sparse_core_info()
_NC, _NS = _info.num_cores, _info.num_subcores
_NW = _NC * _NS
_BPW = BATCH // _NW

_mesh = plsc.VectorSubcoreMesh(core_axis_name="c", subcore_axis_name="s")


@functools.partial(
    pl.kernel,
    mesh=_mesh,
    out_type=jax.ShapeDtypeStruct((BATCH, EMB_DIM), jnp.float32),
    scratch_types=[
        pltpu.VMEM((_BPW,), jnp.int32),
        pltpu.SemaphoreType.DMA,
    ],
)
def _gather_kernel(idx_hbm, table_hbm, out_hbm, idx_v, sem):
    wid = lax.axis_index("s") * _NC + lax.axis_index("c")
    base = wid * _BPW
    pltpu.sync_copy(idx_hbm.at[wid], idx_v)

    @pl.loop(0, _BPW, step=16)
    def _fire(j):
        vec = idx_v[pl.ds(j, 16)]
        for k in range(16):
            row = vec[k]
            pltpu.make_async_copy(
                table_hbm.at[pl.ds(row, 1), :],
                out_hbm.at[pl.ds(base + j + k, 1), :],
                sem,
            ).start()

    pltpu.make_async_copy(
        table_hbm.at[pl.ds(0, _BPW), :],
        out_hbm.at[pl.ds(base, _BPW), :],
        sem,
    ).wait()


def kernel(context_ids, table):
    idx = context_ids.astype(jnp.int32).reshape(_NW, _BPW)
    return _gather_kernel(idx, table)


# speedup vs baseline: 4.1207x; 4.1207x over previous
import functools

import jax
import jax.numpy as jnp
from jax import lax
from jax.experimental import pallas as pl
from jax.experimental.pallas import tpu as pltpu
from jax.experimental.pallas import tpu_sc as plsc

NUM_CONTEXTS = 1000000
EMB_DIM = 32
BATCH = 16384

_info = plsc.get_sparse_core_info()
_NC, _NS = _info.num_cores, _info.num_subcores
_NW = _NC * _NS
_BPW = BATCH // _NW
_LANES = 16
_G = 8
_NPAIR = _BPW // (2 * _G)

_mesh = plsc.VectorSubcoreMesh(core_axis_name="c", subcore_axis_name="s")

_J_LO = tuple(range(16))
_J_HI = tuple(range(16, 32))


@functools.partial(
    pl.kernel,
    mesh=_mesh,
    compiler_params=pltpu.CompilerParams(needs_layout_passes=False),
    out_type=jax.ShapeDtypeStruct((EMB_DIM, BATCH), jnp.float32),
    scratch_types=[
        pltpu.VMEM((_BPW,), jnp.int32),
        pltpu.VMEM((2 * _G, EMB_DIM, 128), jnp.float32),
        pltpu.VMEM((EMB_DIM, _BPW), jnp.float32),
        pltpu.SemaphoreType.DMA,
        pltpu.SemaphoreType.DMA,
    ],
)
def _gather_kernel(idx_hbm, table_t_hbm, out_t_hbm, idx_v, blk_v, cols_v,
                   sem_a, sem_b):
    wid = lax.axis_index("s") * _NC + lax.axis_index("c")
    base = wid * _BPW
    pltpu.sync_copy(idx_hbm.at[wid], idx_v)

    j_lo = lax.broadcasted_iota(jnp.int32, (_LANES,), 0)
    j_hi = j_lo + _LANES

    def _fire(c_scalar, slot, sem):
        col = pl.multiple_of(c_scalar * 128, 128)
        pltpu.make_async_copy(
            table_t_hbm.at[:, pl.ds(col, 128)],
            blk_v.at[slot],
            sem,
        ).start()

    def _drain(slot, sem):
        pltpu.make_async_copy(
            table_t_hbm.at[:, pl.ds(0, 128)],
            blk_v.at[slot],
            sem,
        ).wait()

    def _extract(slot, m_scalar, pos):
        mvec = jnp.full((_LANES,), m_scalar, dtype=jnp.int32)
        svec = jnp.full((_LANES,), slot, dtype=jnp.int32)
        pvec = jnp.full((_LANES,), pos, dtype=jnp.int32)
        lo = plsc.load_gather(blk_v, [svec, j_lo, mvec])
        hi = plsc.load_gather(blk_v, [svec, j_hi, mvec])
        plsc.store_scatter(cols_v, [j_lo, pvec], lo)
        plsc.store_scatter(cols_v, [j_hi, pvec], hi)

    v0 = idx_v[pl.ds(0, _LANES)]
    c0 = v0 >> 7
    for k in range(_G):
        _fire(c0[k], k, sem_a)

    @pl.loop(0, _NPAIR)
    def _body(t):
        vec = idx_v[pl.ds(t * 2 * _G, _LANES)]
        cvec = vec >> 7
        mvec = vec & 127
        for k in range(_G):
            _fire(cvec[_G + k], _G + k, sem_b)
        for k in range(_G):
            _drain(k, sem_a)
        for k in range(_G):
            _extract(k, mvec[k], t * 2 * _G + k)
        @pl.when(t < _NPAIR - 1)
        def _():
            vnext = idx_v[pl.ds((t + 1) * 2 * _G, _LANES)]
            cnext = vnext >> 7
            for k in range(_G):
                _fire(cnext[k], k, sem_a)
        for k in range(_G):
            _drain(_G + k, sem_b)
        for k in range(_G):
            _extract(_G + k, mvec[_G + k], t * 2 * _G + _G + k)

    pltpu.sync_copy(cols_v, out_t_hbm.at[:, pl.ds(base, _BPW)])


def kernel(context_ids, table):
    idx = context_ids.astype(jnp.int32).reshape(_NW, _BPW)
    table_t = jnp.transpose(table)
    out_t = _gather_kernel(idx, table_t)
    return jnp.transpose(out_t)


# speedup vs baseline: 4.1302x; 1.0023x over previous
import functools

import jax
import jax.numpy as jnp
from jax import lax
from jax.experimental import pallas as pl
from jax.experimental.pallas import tpu as pltpu
from jax.experimental.pallas import tpu_sc as plsc

NUM_CONTEXTS = 1000000
EMB_DIM = 32
BATCH = 16384

_info = plsc.get_sparse_core_info()
_NC, _NS = _info.num_cores, _info.num_subcores
_NW = _NC * _NS
_BPW = BATCH // _NW
_LANES = 16
_G = 8
_NPAIR = _BPW // (2 * _G)

_mesh = plsc.VectorSubcoreMesh(core_axis_name="c", subcore_axis_name="s")


@functools.partial(
    pl.kernel,
    mesh=_mesh,
    compiler_params=pltpu.CompilerParams(needs_layout_passes=False),
    out_type=jax.ShapeDtypeStruct((EMB_DIM, BATCH), jnp.float32),
    scratch_types=[
        pltpu.VMEM((_BPW,), jnp.int32),
        pltpu.VMEM((2 * _G, EMB_DIM, 128), jnp.float32),
        pltpu.VMEM((EMB_DIM, _BPW), jnp.float32),
        pltpu.SemaphoreType.DMA,
        pltpu.SemaphoreType.DMA,
    ],
)
def _gather_kernel(idx_hbm, table_t_hbm, out_t_hbm, idx_v, blk_v, cols_v,
                   sem_a, sem_b):
    wid = lax.axis_index("s") * _NC + lax.axis_index("c")
    base = wid * _BPW
    pltpu.sync_copy(idx_hbm.at[wid], idx_v)

    j_lo = lax.broadcasted_iota(jnp.int32, (_LANES,), 0)
    j_hi = j_lo + _LANES

    def _fire(c_scalar, slot, sem):
        col = pl.multiple_of(c_scalar * 128, 128)
        pltpu.make_async_copy(
            table_t_hbm.at[:, pl.ds(col, 128)],
            blk_v.at[slot],
            sem,
        ).start()

    def _drain(slot, sem):
        pltpu.make_async_copy(
            table_t_hbm.at[:, pl.ds(0, 128)],
            blk_v.at[slot],
            sem,
        ).wait()

    def _extract(slot, m_scalar, pos):
        mvec = jnp.full((_LANES,), m_scalar, dtype=jnp.int32)
        svec = jnp.full((_LANES,), slot, dtype=jnp.int32)
        pvec = jnp.full((_LANES,), pos, dtype=jnp.int32)
        lo = plsc.load_gather(blk_v, [svec, j_lo, mvec])
        hi = plsc.load_gather(blk_v, [svec, j_hi, mvec])
        plsc.store_scatter(cols_v, [j_lo, pvec], lo)
        plsc.store_scatter(cols_v, [j_hi, pvec], hi)

    v0 = idx_v[pl.ds(0, _LANES)]
    c0 = v0 >> 7
    for k in range(_G):
        _fire(c0[k], k, sem_a)

    @pl.loop(0, _NPAIR)
    def _body(t):
        vec = idx_v[pl.ds(t * 2 * _G, _LANES)]
        cvec = vec >> 7
        mvec = vec & 127
        for k in range(_G):
            _fire(cvec[_G + k], _G + k, sem_b)
        for k in range(_G):
            _drain(k, sem_a)
        for k in range(_G):
            _extract(k, mvec[k], t * 2 * _G + k)
        @pl.when(t < _NPAIR - 1)
        def _():
            vnext = idx_v[pl.ds((t + 1) * 2 * _G, _LANES)]
            cnext = vnext >> 7
            for k in range(_G):
                _fire(cnext[k], k, sem_a)
        for k in range(_G):
            _drain(_G + k, sem_b)
        for k in range(_G):
            _extract(_G + k, mvec[_G + k], t * 2 * _G + _G + k)

    pltpu.sync_copy(cols_v, out_t_hbm.at[:, pl.ds(base, _BPW)])


def kernel(context_ids, table):
    idx = context_ids.astype(jnp.int32).reshape(_NW, _BPW)
    table_t = jnp.transpose(table)
    out_t = _gather_kernel(idx, table_t)
    return jnp.transpose(out_t)
